# baseline (speedup 1.0000x reference)
import functools

import jax
import jax.numpy as jnp
from jax import lax
from jax.experimental import pallas as pl
from jax.experimental.pallas import tpu as pltpu
from jax.experimental.pallas import tpu_sc as plsc

EMBED = 64
IMG_TOK = 151669
BATCH = 16
SEQ = 2048
TOT = BATCH * SEQ
VOCAB_ROWS = 200000
HALF = VOCAB_ROWS // 2

NC, NS = 2, 16
NW = NC * NS
ROWS_PER_W = TOT // NW
L = 16

K1_CHUNK = 3200


def _mesh():
    return plsc.VectorSubcoreMesh(
        core_axis_name="c", subcore_axis_name="s", num_cores=NC, num_subcores=NS
    )


@functools.cache
def _build_sc_repack():
    @functools.partial(
        pl.kernel,
        mesh=_mesh(),
        out_type=jax.ShapeDtypeStruct((HALF, 2 * EMBED), jnp.float32),
        scratch_types=[
            pltpu.VMEM((256, EMBED), jnp.float32),
            pltpu.VMEM((256, EMBED), jnp.float32),
            pltpu.VMEM((128, 2 * EMBED), jnp.float32),
            pltpu.VMEM((128, 2 * EMBED), jnp.float32),
            pltpu.SemaphoreType.DMA,
            pltpu.SemaphoreType.DMA,
        ],
    )
    def _sc_repack(table, packed, inA, inB, outA, outB, semA, semB):
        wid = lax.axis_index("s") * NC + lax.axis_index("c")
        row0 = wid * K1_CHUNK

        def fire(prow, rows, buf, sem):
            pltpu.async_copy(
                table.at[pl.ds(2 * prow, 2 * rows)], buf.at[pl.ds(0, 2 * rows)],
                sem,
            )

        def wait(rows, buf, sem):
            pltpu.make_async_copy(
                table.at[pl.ds(0, 2 * rows)], buf.at[pl.ds(0, 2 * rows)], sem
            ).wait()

        def assemble(rows, src, dst):
            def row(j, _):
                for q in range(4):
                    dst[j, pl.ds(q * L, L)] = src[2 * j, pl.ds(q * L, L)]
                    dst[j, pl.ds(EMBED + q * L, L)] = src[
                        2 * j + 1, pl.ds(q * L, L)
                    ]
                return 0

            lax.fori_loop(0, rows, row, 0)

        def make_dstep(rows):
            def dstep(k, _):
                r0 = row0 + k * 2 * rows
                fire(r0, rows, inA, semA)
                fire(r0 + rows, rows, inB, semB)
                wait(rows, inA, semA)
                assemble(rows, inA, outA)
                pltpu.sync_copy(
                    outA.at[pl.ds(0, rows)], packed.at[pl.ds(r0, rows)]
                )
                wait(rows, inB, semB)
                assemble(rows, inB, outB)
                pltpu.sync_copy(
                    outB.at[pl.ds(0, rows)], packed.at[pl.ds(r0 + rows, rows)]
                )
                return 0

            return dstep

        n = jnp.where(wid == NW - 1, 5, K1_CHUNK // 160)
        lax.fori_loop(0, n, make_dstep(80), 0)

    return _sc_repack


@functools.cache
def _build_sc_gather():
    @functools.partial(
        pl.kernel,
        mesh=_mesh(),
        out_type=jax.ShapeDtypeStruct((TOT, EMBED), jnp.float32),
        scratch_types=[
            pltpu.VMEM((8, 128), jnp.int32),
            pltpu.VMEM((ROWS_PER_W, EMBED), jnp.float32),
            pltpu.SemaphoreType.DMA,
        ],
        compiler_params=pltpu.CompilerParams(use_tc_tiling_on_sc=False),
    )
    def _sc_gather(table_lin, ids2d, out_hbm, idx_v, rows_v, sem):
        wid = lax.axis_index("s") * NC + lax.axis_index("c")
        pltpu.sync_copy(ids2d.at[pl.ds(wid * 8, 8)], idx_v)

        copies = []
        for j in range(8):
            copies.append(
                pltpu.async_copy(
                    table_lin.at[idx_v.at[j]],
                    rows_v.at[pl.ds(j * 128, 128)],
                    sem,
                )
            )
        for c in copies:
            c.wait()
        pltpu.sync_copy(rows_v, out_hbm.at[pl.ds(wid * ROWS_PER_W, ROWS_PER_W)])

    return _sc_gather


def _mask_body(ids_ref, attn_ref, am_out, im_out):
    am_out[...] = attn_ref[...] == 1
    im_out[...] = ids_ref[...] == IMG_TOK


def _masks_tc(input_ids, attention_mask):
    return pl.pallas_call(
        _mask_body,
        out_shape=(
            jax.ShapeDtypeStruct((BATCH, SEQ), jnp.bool_),
            jax.ShapeDtypeStruct((BATCH, SEQ), jnp.bool_),
        ),
    )(input_ids, attention_mask)


def kernel(pixel_values, input_ids, attention_mask, text_proj_weight):
    del pixel_values
    ids32 = input_ids.astype(jnp.int32)
    packed = _build_sc_repack()(text_proj_weight)
    table_lin = packed.reshape(VOCAB_ROWS, EMBED)
    ids2d = ids32.reshape(TOT // 128, 128)
    flat = _build_sc_gather()(table_lin, ids2d)
    hidden_states = flat.reshape(BATCH, SEQ, EMBED)
    attn_mask, image_mask = _masks_tc(ids32, attention_mask.astype(jnp.int32))
    return (hidden_states, attn_mask, image_mask)


# speedup vs baseline: 1.5897x; 1.5897x over previous
import functools

import jax
import jax.numpy as jnp
from jax import lax
from jax.experimental import pallas as pl
from jax.experimental.pallas import tpu as pltpu
from jax.experimental.pallas import tpu_sc as plsc

EMBED = 64
IMG_TOK = 151669
BATCH = 16
SEQ = 2048
TOT = BATCH * SEQ

NC, NS = 2, 16
NW = NC * NS
ROWS_PER_W = TOT // NW
IDX_CHUNK = 128
CH = ROWS_PER_W // IDX_CHUNK

@functools.cache
def _build_sc_gather():
    mesh = plsc.VectorSubcoreMesh(
        core_axis_name="c", subcore_axis_name="s", num_cores=NC, num_subcores=NS
    )

    @functools.partial(
        pl.kernel,
        mesh=mesh,
        out_type=jax.ShapeDtypeStruct((BATCH, SEQ, EMBED), jnp.float32),
        scratch_types=[
            pltpu.VMEM((CH, IDX_CHUNK), jnp.int32),
            pltpu.VMEM((ROWS_PER_W, EMBED), jnp.float32),
            pltpu.SemaphoreType.DMA,
        ],
        compiler_params=pltpu.CompilerParams(use_tc_tiling_on_sc=False),
    )
    def _sc_gather(table_hbm, ids_hbm, out_hbm, idx_v, rows_v, sem):
        wid = lax.axis_index("s") * NC + lax.axis_index("c")
        pltpu.sync_copy(ids_hbm.at[pl.ds(wid * CH, CH)], idx_v)
        copies = []
        for j in range(CH):
            copies.append(
                pltpu.async_copy(
                    table_hbm.at[idx_v.at[j]],
                    rows_v.at[pl.ds(j * IDX_CHUNK, IDX_CHUNK)],
                    sem,
                )
            )
        for c in copies:
            c.wait()
        pltpu.sync_copy(
            rows_v,
            out_hbm.at[wid // 2, pl.ds((wid % 2) * ROWS_PER_W, ROWS_PER_W)],
        )

    return _sc_gather


def _mask_body(ids_ref, attn_ref, am_out, im_out):
    am_out[...] = attn_ref[...] == 1
    im_out[...] = ids_ref[...] == IMG_TOK


def _masks_tc(input_ids, attention_mask):
    return pl.pallas_call(
        _mask_body,
        out_shape=(
            jax.ShapeDtypeStruct((BATCH, SEQ), jnp.bool_),
            jax.ShapeDtypeStruct((BATCH, SEQ), jnp.bool_),
        ),
    )(input_ids, attention_mask)


def kernel(pixel_values, input_ids, attention_mask, text_proj_weight):
    del pixel_values
    ids32 = input_ids.astype(jnp.int32)
    ids_tiled = ids32.reshape(TOT // IDX_CHUNK, IDX_CHUNK)
    hidden_states = _build_sc_gather()(text_proj_weight, ids_tiled)
    attn_mask, image_mask = _masks_tc(ids32, attention_mask.astype(jnp.int32))
    return (hidden_states, attn_mask, image_mask)
